# baseline (speedup 1.0000x reference)
import jax
import jax.numpy as jnp
from jax import lax
from jax.experimental import pallas as pl
from jax.experimental.pallas import tpu as pltpu
from jax.experimental.pallas import tpu_sc as plsc

BATCH = 4096
SEQ_LEN = 200
EMBED_DIM = 64
VOCAB = 1000000
ROW_PAD = 128

NUM_CORES = 2
NUM_SUBCORES = 16
NUM_WORKERS = NUM_CORES * NUM_SUBCORES

BLOCK_B = BATCH // NUM_WORKERS
NBUF = 2
LANES = 16
GROUPS = EMBED_DIM // LANES
OPITCH = 137


def _wid():
    return lax.axis_index("s") * NUM_CORES + lax.axis_index("c")


def _embed_body(idx_hbm, table_hbm, pos_hbm, out_hbm, idx_v, rows_bufs,
                out_bufs, pos_v, gsems, wsems):
    wid = _wid()
    b0 = wid * BLOCK_B

    pltpu.sync_copy(pos_hbm, pos_v)
    pltpu.sync_copy(idx_hbm.at[:, pl.ds(b0, BLOCK_B)], idx_v)

    iota = lax.iota(jnp.int32, LANES)
    ge = [iota + be for be in range(0, EMBED_DIM, LANES)]

    def gather_copy(b, l):
        return pltpu.make_async_copy(
            table_hbm.at[idx_v.at[l]], rows_bufs[b], gsems[b])

    def write_copy(b, l):
        return pltpu.make_async_copy(
            out_bufs[b].at[:, pl.ds(0, BLOCK_B)],
            out_hbm.at[l, :, pl.ds(b0, BLOCK_B)], wsems[b])

    def transpose_add(b, l):
        pos_g = [pos_v[l, pl.ds(g * LANES, LANES)] for g in range(GROUPS)]

        def bbody(bb, carry):
            col_idx = jnp.broadcast_to(bb, (LANES,)).astype(jnp.int32)
            for g in range(GROUPS):
                v = rows_bufs[b][bb, pl.ds(g * LANES, LANES)] + pos_g[g]
                plsc.store_scatter(out_bufs[b], [ge[g], col_idx], v)
            return carry

        lax.fori_loop(0, BLOCK_B, bbody, 0, unroll=False)

    for b in range(NBUF - 1):
        gather_copy(b, b).start()

    def outer(o, carry):
        for b in range(NBUF):
            l = o * NBUF + b
            gather_copy(b, l).wait()

            @pl.when(l >= NBUF)
            def _():
                write_copy(b, l - NBUF).wait()

            transpose_add(b, l)

            @pl.when(l + NBUF - 1 <= SEQ_LEN - 1)
            def _():
                gather_copy((b - 1) % NBUF, l + NBUF - 1).start()

            write_copy(b, l).start()
        return carry

    lax.fori_loop(0, SEQ_LEN // NBUF, outer, 0, unroll=False)

    for l in range(SEQ_LEN - SEQ_LEN % NBUF, SEQ_LEN):
        b = l % NBUF
        gather_copy(b, l).wait()
        write_copy(b, l - NBUF).wait()
        transpose_add(b, l)
        write_copy(b, l).start()

    for l in range(SEQ_LEN - NBUF, SEQ_LEN):
        write_copy(l % NBUF, l).wait()


@jax.jit
def _embed(inputs, token_table, pos_table):
    mesh = plsc.VectorSubcoreMesh(
        core_axis_name="c", subcore_axis_name="s", num_cores=NUM_CORES,
        num_subcores=NUM_SUBCORES)

    table_p = jnp.pad(token_table, ((0, 0), (0, ROW_PAD - EMBED_DIM)))
    idx_t = jnp.transpose(inputs)

    f = pl.kernel(
        _embed_body,
        out_type=jax.ShapeDtypeStruct((SEQ_LEN, EMBED_DIM, BATCH),
                                      jnp.float32),
        mesh=mesh,
        scratch_types=[
            pltpu.VMEM((SEQ_LEN, BLOCK_B), jnp.int32),
            [pltpu.VMEM((BLOCK_B, ROW_PAD), jnp.float32)] * NBUF,
            [pltpu.VMEM((EMBED_DIM, OPITCH), jnp.float32)] * NBUF,
            pltpu.VMEM((SEQ_LEN, EMBED_DIM), jnp.float32),
            [pltpu.SemaphoreType.DMA] * NBUF,
            [pltpu.SemaphoreType.DMA] * NBUF,
        ],
        compiler_params=pltpu.CompilerParams(use_tc_tiling_on_sc=True,
                                             needs_layout_passes=False),
    )
    out_t = f(idx_t, table_p, pos_table)
    return jnp.transpose(out_t, (2, 0, 1))


def kernel(inputs, token_table, pos_table):
    return _embed(inputs, token_table, pos_table)


# speedup vs baseline: 1.8076x; 1.8076x over previous
import jax
import jax.numpy as jnp
from jax import lax
from jax.experimental import pallas as pl
from jax.experimental.pallas import tpu as pltpu
from jax.experimental.pallas import tpu_sc as plsc

BATCH = 4096
SEQ_LEN = 200
EMBED_DIM = 64
VOCAB = 1000000
ROW_PAD = 128

NUM_CORES = 2
NUM_SUBCORES = 16
NUM_WORKERS = NUM_CORES * NUM_SUBCORES

BLOCK_B = BATCH // NUM_WORKERS
NBUF = 3
LANES = 16
GROUPS = EMBED_DIM // LANES
OPITCH = 137


def _wid():
    return lax.axis_index("s") * NUM_CORES + lax.axis_index("c")


def _embed_body(idx_hbm, table_hbm, pos_hbm, out_hbm, idx_v, rows_bufs,
                out_bufs, pos_v, gsems, wsems):
    wid = _wid()
    b0 = wid * BLOCK_B

    pltpu.sync_copy(pos_hbm, pos_v)
    pltpu.sync_copy(idx_hbm.at[:, pl.ds(b0, BLOCK_B)], idx_v)

    iota = lax.iota(jnp.int32, LANES)
    rot = [jnp.bitwise_and(iota + k, LANES - 1) for k in range(LANES)]
    ge = [iota + be for be in range(0, EMBED_DIM, LANES)]

    def gather_copy(b, l):
        return pltpu.make_async_copy(
            table_hbm.at[idx_v.at[l]], rows_bufs[b], gsems[b])

    def write_copy(b, l):
        return pltpu.make_async_copy(
            out_bufs[b], out_hbm.at[l, :, pl.ds(b0, BLOCK_B)], wsems[b])

    def transpose_add(b, l):
        pos_g = [pos_v[l, pl.ds(g * LANES, LANES)] for g in range(GROUPS)]

        @plsc.parallel_loop(0, BLOCK_B // LANES, 1, unroll=1)
        def tbody(bt8):
            bbv = jnp.broadcast_to(bt8 * LANES, (LANES,)).astype(jnp.int32)
            for k in range(LANES):
                bcol = bbv + rot[k]
                for g in range(GROUPS):
                    v = plsc.load_gather(rows_bufs[b], [bcol, ge[g]])
                    plsc.store_scatter(out_bufs[b], [ge[g], bcol],
                                       v + pos_g[g])

    for b in range(NBUF - 1):
        gather_copy(b, b).start()

    def outer(o, carry):
        for b in range(NBUF):
            l = o * NBUF + b
            gather_copy(b, l).wait()

            @pl.when(l >= NBUF)
            def _():
                write_copy(b, l - NBUF).wait()

            transpose_add(b, l)

            @pl.when(l + NBUF - 1 <= SEQ_LEN - 1)
            def _():
                gather_copy((b - 1) % NBUF, l + NBUF - 1).start()

            write_copy(b, l).start()
        return carry

    lax.fori_loop(0, SEQ_LEN // NBUF, outer, 0, unroll=False)

    for l in range(SEQ_LEN - SEQ_LEN % NBUF, SEQ_LEN):
        b = l % NBUF
        gather_copy(b, l).wait()
        write_copy(b, l - NBUF).wait()
        transpose_add(b, l)
        write_copy(b, l).start()

    for l in range(SEQ_LEN - NBUF, SEQ_LEN):
        write_copy(l % NBUF, l).wait()


@jax.jit
def _embed(inputs, token_table, pos_table):
    mesh = plsc.VectorSubcoreMesh(
        core_axis_name="c", subcore_axis_name="s", num_cores=NUM_CORES,
        num_subcores=NUM_SUBCORES)

    table_p = jnp.pad(token_table, ((0, 0), (0, ROW_PAD - EMBED_DIM)))
    idx_t = jnp.transpose(inputs)

    f = pl.kernel(
        _embed_body,
        out_type=jax.ShapeDtypeStruct((SEQ_LEN, EMBED_DIM, BATCH),
                                      jnp.float32),
        mesh=mesh,
        scratch_types=[
            pltpu.VMEM((SEQ_LEN, BLOCK_B), jnp.int32),
            [pltpu.VMEM((BLOCK_B, ROW_PAD), jnp.float32)] * NBUF,
            [pltpu.VMEM((EMBED_DIM, BLOCK_B), jnp.float32)] * NBUF,
            pltpu.VMEM((SEQ_LEN, EMBED_DIM), jnp.float32),
            [pltpu.SemaphoreType.DMA] * NBUF,
            [pltpu.SemaphoreType.DMA] * NBUF,
        ],
        compiler_params=pltpu.CompilerParams(use_tc_tiling_on_sc=True,
                                             needs_layout_passes=False),
    )
    out_t = f(idx_t, table_p, pos_table)
    return jnp.transpose(out_t, (2, 0, 1))


def kernel(inputs, token_table, pos_table):
    return _embed(inputs, token_table, pos_table)


# speedup vs baseline: 1.8953x; 1.0485x over previous
import jax
import jax.numpy as jnp
from jax import lax
from jax.experimental import pallas as pl
from jax.experimental.pallas import tpu as pltpu
from jax.experimental.pallas import tpu_sc as plsc

BATCH = 4096
SEQ_LEN = 200
EMBED_DIM = 64
VOCAB = 1000000
ROW_PAD = 128

NUM_CORES = 2
NUM_SUBCORES = 16
NUM_WORKERS = NUM_CORES * NUM_SUBCORES

BLOCK_B = BATCH // NUM_WORKERS
NBUF = 3
LANES = 16
GROUPS = EMBED_DIM // LANES
OPITCH = 137


TILE_C = 128
NUM_TCOLS = (VOCAB + TILE_C - 1) // TILE_C
COLS_PER_W = (NUM_TCOLS + NUM_WORKERS - 1) // NUM_WORKERS
TROWS = NUM_TCOLS * TILE_C
RBUF = 3


def _wid():
    return lax.axis_index("s") * NUM_CORES + lax.axis_index("c")


def _repack_body(tab_t_hbm, out_hbm, stage_bufs, trans_bufs, gsems, wsems):
    wid = _wid()
    iota = lax.iota(jnp.int32, LANES)
    rot = [jnp.bitwise_and(iota + k, LANES - 1) for k in range(LANES)]
    ge = [iota + be for be in range(0, EMBED_DIM, LANES)]

    def col(k):
        return wid + k * NUM_WORKERS

    def read_copy(b, k):
        return pltpu.make_async_copy(
            tab_t_hbm.at[:, pl.ds(col(k) * TILE_C, TILE_C)], stage_bufs[b],
            gsems[b])

    def write_copy(b, k):
        return pltpu.make_async_copy(
            trans_bufs[b], out_hbm.at[pl.ds(col(k) * TILE_C, TILE_C)],
            wsems[b])

    def transpose(b):
        @plsc.parallel_loop(0, TILE_C // LANES, 1, unroll=1)
        def tbody(bt8):
            btv = jnp.broadcast_to(bt8 * LANES, (LANES,)).astype(jnp.int32)
            for k in range(LANES):
                tok = btv + rot[k]
                for g in range(GROUPS):
                    v = plsc.load_gather(stage_bufs[b], [ge[g], tok])
                    plsc.store_scatter(trans_bufs[b], [tok, ge[g]], v)

    for b in range(RBUF - 1):
        @pl.when(col(b) < NUM_TCOLS)
        def _():
            read_copy(b, b).start()

    def outer(o, carry):
        for b in range(RBUF):
            k = o * RBUF + b
            c = col(k)

            @pl.when(c < NUM_TCOLS)
            def _():
                read_copy(b, k).wait()

                @pl.when(k >= RBUF)
                def _():
                    write_copy(b, k - RBUF).wait()

                transpose(b)

            @pl.when(col(k + RBUF - 1) < NUM_TCOLS)
            def _():
                read_copy((b - 1) % RBUF, k + RBUF - 1).start()

            @pl.when(c < NUM_TCOLS)
            def _():
                write_copy(b, k).start()
        return carry

    n_outer = COLS_PER_W // RBUF
    lax.fori_loop(0, n_outer, outer, 0, unroll=False)

    for k in range(n_outer * RBUF, COLS_PER_W):
        b = k % RBUF
        write_copy(b, k - RBUF).wait()

        @pl.when(col(k) < NUM_TCOLS)
        def _():
            read_copy(b, k).wait()
            transpose(b)
            write_copy(b, k).start()

    for k in range(COLS_PER_W - RBUF, COLS_PER_W):
        b = k % RBUF
        if k < n_outer * RBUF:
            write_copy(b, k).wait()
        else:
            @pl.when(col(k) < NUM_TCOLS)
            def _():
                write_copy(b, k).wait()


def _embed_body(idx_hbm, table_hbm, pos_hbm, out_hbm, idx_v, rows_bufs,
                out_bufs, pos_v, gsems, wsems):
    wid = _wid()
    b0 = wid * BLOCK_B

    pltpu.sync_copy(pos_hbm, pos_v)
    pltpu.sync_copy(idx_hbm.at[:, pl.ds(b0, BLOCK_B)], idx_v)

    iota = lax.iota(jnp.int32, LANES)
    rot = [jnp.bitwise_and(iota + k, LANES - 1) for k in range(LANES)]
    ge = [iota + be for be in range(0, EMBED_DIM, LANES)]

    def gather_copy(b, l):
        return pltpu.make_async_copy(
            table_hbm.at[idx_v.at[l]], rows_bufs[b], gsems[b])

    def write_copy(b, l):
        return pltpu.make_async_copy(
            out_bufs[b], out_hbm.at[l, :, pl.ds(b0, BLOCK_B)], wsems[b])

    def transpose_add(b, l):
        pos_g = [pos_v[l, pl.ds(g * LANES, LANES)] for g in range(GROUPS)]

        @plsc.parallel_loop(0, BLOCK_B // LANES, 1, unroll=1)
        def tbody(bt8):
            bbv = jnp.broadcast_to(bt8 * LANES, (LANES,)).astype(jnp.int32)
            for k in range(LANES):
                bcol = bbv + rot[k]
                for g in range(GROUPS):
                    v = plsc.load_gather(rows_bufs[b], [bcol, ge[g]])
                    plsc.store_scatter(out_bufs[b], [ge[g], bcol],
                                       v + pos_g[g])

    for b in range(NBUF - 1):
        gather_copy(b, b).start()

    def outer(o, carry):
        for b in range(NBUF):
            l = o * NBUF + b
            gather_copy(b, l).wait()

            @pl.when(l >= NBUF)
            def _():
                write_copy(b, l - NBUF).wait()

            transpose_add(b, l)

            @pl.when(l + NBUF - 1 <= SEQ_LEN - 1)
            def _():
                gather_copy((b - 1) % NBUF, l + NBUF - 1).start()

            write_copy(b, l).start()
        return carry

    lax.fori_loop(0, SEQ_LEN // NBUF, outer, 0, unroll=False)

    for l in range(SEQ_LEN - SEQ_LEN % NBUF, SEQ_LEN):
        b = l % NBUF
        gather_copy(b, l).wait()
        write_copy(b, l - NBUF).wait()
        transpose_add(b, l)
        write_copy(b, l).start()

    for l in range(SEQ_LEN - NBUF, SEQ_LEN):
        write_copy(l % NBUF, l).wait()


@jax.jit
def _embed(inputs, token_table, pos_table):
    mesh = plsc.VectorSubcoreMesh(
        core_axis_name="c", subcore_axis_name="s", num_cores=NUM_CORES,
        num_subcores=NUM_SUBCORES)

    repack = pl.kernel(
        _repack_body,
        out_type=jax.ShapeDtypeStruct((TROWS, ROW_PAD), jnp.float32),
        mesh=mesh,
        scratch_types=[
            [pltpu.VMEM((EMBED_DIM, TILE_C), jnp.float32)] * RBUF,
            [pltpu.VMEM((TILE_C, ROW_PAD), jnp.float32)] * RBUF,
            [pltpu.SemaphoreType.DMA] * RBUF,
            [pltpu.SemaphoreType.DMA] * RBUF,
        ],
        compiler_params=pltpu.CompilerParams(use_tc_tiling_on_sc=True,
                                             needs_layout_passes=False),
    )
    table_p = repack(jnp.transpose(token_table))
    idx_t = jnp.transpose(inputs)

    f = pl.kernel(
        _embed_body,
        out_type=jax.ShapeDtypeStruct((SEQ_LEN, EMBED_DIM, BATCH),
                                      jnp.float32),
        mesh=mesh,
        scratch_types=[
            pltpu.VMEM((SEQ_LEN, BLOCK_B), jnp.int32),
            [pltpu.VMEM((BLOCK_B, ROW_PAD), jnp.float32)] * NBUF,
            [pltpu.VMEM((EMBED_DIM, BLOCK_B), jnp.float32)] * NBUF,
            pltpu.VMEM((SEQ_LEN, EMBED_DIM), jnp.float32),
            [pltpu.SemaphoreType.DMA] * NBUF,
            [pltpu.SemaphoreType.DMA] * NBUF,
        ],
        compiler_params=pltpu.CompilerParams(use_tc_tiling_on_sc=True,
                                             needs_layout_passes=False),
    )
    out_t = f(idx_t, table_p, pos_table)
    return jnp.transpose(out_t, (2, 0, 1))


def kernel(inputs, token_table, pos_table):
    return _embed(inputs, token_table, pos_table)


# speedup vs baseline: 1.9133x; 1.0095x over previous
import jax
import jax.numpy as jnp
from jax import lax
from jax.experimental import pallas as pl
from jax.experimental.pallas import tpu as pltpu
from jax.experimental.pallas import tpu_sc as plsc

BATCH = 4096
SEQ_LEN = 200
EMBED_DIM = 64
VOCAB = 1000000
ROW_PAD = 128

NUM_CORES = 2
NUM_SUBCORES = 16
NUM_WORKERS = NUM_CORES * NUM_SUBCORES

BLOCK_B = BATCH // NUM_WORKERS
NBUF = 3
LANES = 16
GROUPS = EMBED_DIM // LANES
OPITCH = 137


TILE_C = 128
NUM_TCOLS = (VOCAB + TILE_C - 1) // TILE_C
COLS_PER_W = (NUM_TCOLS + NUM_WORKERS - 1) // NUM_WORKERS
TROWS = NUM_TCOLS * TILE_C
RBUF = 4


def _wid():
    return lax.axis_index("s") * NUM_CORES + lax.axis_index("c")


def _repack_body(tab_t_hbm, out_hbm, stage_bufs, trans_bufs, gsems, wsems):
    wid = _wid()
    iota = lax.iota(jnp.int32, LANES)
    rot = [jnp.bitwise_and(iota + k, LANES - 1) for k in range(LANES)]
    ge = [iota + be for be in range(0, EMBED_DIM, LANES)]

    def col(k):
        return wid + k * NUM_WORKERS

    def read_copy(b, k):
        return pltpu.make_async_copy(
            tab_t_hbm.at[:, pl.ds(col(k) * TILE_C, TILE_C)], stage_bufs[b],
            gsems[b])

    def write_copy(b, k):
        return pltpu.make_async_copy(
            trans_bufs[b], out_hbm.at[pl.ds(col(k) * TILE_C, TILE_C)],
            wsems[b])

    def transpose(b):
        @plsc.parallel_loop(0, TILE_C // LANES, 1, unroll=1)
        def tbody(bt8):
            btv = jnp.broadcast_to(bt8 * LANES, (LANES,)).astype(jnp.int32)
            for k in range(LANES):
                tok = btv + rot[k]
                for g in range(GROUPS):
                    v = plsc.load_gather(stage_bufs[b], [ge[g], tok])
                    plsc.store_scatter(trans_bufs[b], [tok, ge[g]], v)

    for b in range(RBUF - 1):
        @pl.when(col(b) < NUM_TCOLS)
        def _():
            read_copy(b, b).start()

    def outer(o, carry):
        for b in range(RBUF):
            k = o * RBUF + b
            c = col(k)

            @pl.when(c < NUM_TCOLS)
            def _():
                read_copy(b, k).wait()

                @pl.when(k >= RBUF)
                def _():
                    write_copy(b, k - RBUF).wait()

                transpose(b)

            @pl.when(col(k + RBUF - 1) < NUM_TCOLS)
            def _():
                read_copy((b - 1) % RBUF, k + RBUF - 1).start()

            @pl.when(c < NUM_TCOLS)
            def _():
                write_copy(b, k).start()
        return carry

    n_outer = COLS_PER_W // RBUF
    lax.fori_loop(0, n_outer, outer, 0, unroll=False)

    for k in range(n_outer * RBUF, COLS_PER_W):
        b = k % RBUF
        write_copy(b, k - RBUF).wait()

        @pl.when(col(k) < NUM_TCOLS)
        def _():
            read_copy(b, k).wait()
            transpose(b)
            write_copy(b, k).start()

    for k in range(COLS_PER_W - RBUF, COLS_PER_W):
        b = k % RBUF
        if k < n_outer * RBUF:
            write_copy(b, k).wait()
        else:
            @pl.when(col(k) < NUM_TCOLS)
            def _():
                write_copy(b, k).wait()


def _embed_body(idx_hbm, table_hbm, pos_hbm, out_hbm, idx_v, rows_bufs,
                out_bufs, pos_v, gsems, wsems):
    wid = _wid()
    b0 = wid * BLOCK_B

    pltpu.sync_copy(pos_hbm, pos_v)
    pltpu.sync_copy(idx_hbm.at[:, pl.ds(b0, BLOCK_B)], idx_v)

    iota = lax.iota(jnp.int32, LANES)
    rot = [jnp.bitwise_and(iota + k, LANES - 1) for k in range(LANES)]
    ge = [iota + be for be in range(0, EMBED_DIM, LANES)]

    def gather_copy(b, l):
        return pltpu.make_async_copy(
            table_hbm.at[idx_v.at[l]], rows_bufs[b], gsems[b])

    def write_copy(b, l):
        return pltpu.make_async_copy(
            out_bufs[b], out_hbm.at[l, :, pl.ds(b0, BLOCK_B)], wsems[b])

    def transpose_add(b, l):
        pos_g = [pos_v[l, pl.ds(g * LANES, LANES)] for g in range(GROUPS)]

        @plsc.parallel_loop(0, BLOCK_B // LANES, 1, unroll=1)
        def tbody(bt8):
            bbv = jnp.broadcast_to(bt8 * LANES, (LANES,)).astype(jnp.int32)
            for k in range(LANES):
                bcol = bbv + rot[k]
                for g in range(GROUPS):
                    v = plsc.load_gather(rows_bufs[b], [bcol, ge[g]])
                    plsc.store_scatter(out_bufs[b], [ge[g], bcol],
                                       v + pos_g[g])

    for b in range(NBUF - 1):
        gather_copy(b, b).start()

    def outer(o, carry):
        for b in range(NBUF):
            l = o * NBUF + b
            gather_copy(b, l).wait()

            @pl.when(l >= NBUF)
            def _():
                write_copy(b, l - NBUF).wait()

            transpose_add(b, l)

            @pl.when(l + NBUF - 1 <= SEQ_LEN - 1)
            def _():
                gather_copy((b - 1) % NBUF, l + NBUF - 1).start()

            write_copy(b, l).start()
        return carry

    lax.fori_loop(0, SEQ_LEN // NBUF, outer, 0, unroll=False)

    for l in range(SEQ_LEN - SEQ_LEN % NBUF, SEQ_LEN):
        b = l % NBUF
        gather_copy(b, l).wait()
        write_copy(b, l - NBUF).wait()
        transpose_add(b, l)
        write_copy(b, l).start()

    for l in range(SEQ_LEN - NBUF, SEQ_LEN):
        write_copy(l % NBUF, l).wait()


@jax.jit
def _embed(inputs, token_table, pos_table):
    mesh = plsc.VectorSubcoreMesh(
        core_axis_name="c", subcore_axis_name="s", num_cores=NUM_CORES,
        num_subcores=NUM_SUBCORES)

    repack = pl.kernel(
        _repack_body,
        out_type=jax.ShapeDtypeStruct((TROWS, ROW_PAD), jnp.float32),
        mesh=mesh,
        scratch_types=[
            [pltpu.VMEM((EMBED_DIM, TILE_C), jnp.float32)] * RBUF,
            [pltpu.VMEM((TILE_C, ROW_PAD), jnp.float32)] * RBUF,
            [pltpu.SemaphoreType.DMA] * RBUF,
            [pltpu.SemaphoreType.DMA] * RBUF,
        ],
        compiler_params=pltpu.CompilerParams(use_tc_tiling_on_sc=True,
                                             needs_layout_passes=False),
    )
    table_p = repack(jnp.transpose(token_table))
    idx_t = jnp.transpose(inputs)

    f = pl.kernel(
        _embed_body,
        out_type=jax.ShapeDtypeStruct((SEQ_LEN, EMBED_DIM, BATCH),
                                      jnp.float32),
        mesh=mesh,
        scratch_types=[
            pltpu.VMEM((SEQ_LEN, BLOCK_B), jnp.int32),
            [pltpu.VMEM((BLOCK_B, ROW_PAD), jnp.float32)] * NBUF,
            [pltpu.VMEM((EMBED_DIM, BLOCK_B), jnp.float32)] * NBUF,
            pltpu.VMEM((SEQ_LEN, EMBED_DIM), jnp.float32),
            [pltpu.SemaphoreType.DMA] * NBUF,
            [pltpu.SemaphoreType.DMA] * NBUF,
        ],
        compiler_params=pltpu.CompilerParams(use_tc_tiling_on_sc=True,
                                             needs_layout_passes=False),
    )
    out_t = f(idx_t, table_p, pos_table)
    return jnp.transpose(out_t, (2, 0, 1))


def kernel(inputs, token_table, pos_table):
    return _embed(inputs, token_table, pos_table)
